# Initial kernel scaffold; baseline (speedup 1.0000x reference)
import jax
import jax.numpy as jnp
from jax.experimental import pallas as pl


def kernel(loc_preds, loc_targets, cls_preds, cls_targets):
    raise NotImplementedError("write your pallas kernel here")


# speedup vs baseline: 2.0002x; 2.0002x over previous
import functools

import jax
import jax.numpy as jnp
from jax.experimental import pallas as pl


def _ohem_reduce_kernel(lp_ref, lt_ref, ct_ref, s_ref, n_ref, *, anchors_per_row):
    i = pl.program_id(0)

    diff = lp_ref[...] - lt_ref[...]
    absd = jnp.abs(diff)
    sl1 = jnp.where(absd < 1.0, 0.5 * diff * diff, absd - 0.5)

    pos = (ct_ref[...] > 0).astype(jnp.float32)

    reps = 128 // anchors_per_row
    j = jax.lax.broadcasted_iota(jnp.int32, (anchors_per_row, 128), 0)
    l = jax.lax.broadcasted_iota(jnp.int32, (anchors_per_row, 128), 1)
    e = (l // reps == j).astype(jnp.float32)

    maskexp = jax.lax.dot(pos, e, precision=jax.lax.Precision.HIGHEST)

    s_part = jnp.sum(sl1 * maskexp).reshape(1, 1)
    n_part = jnp.sum(pos).reshape(1, 1)

    @pl.when(i == 0)
    def _init():
        s_ref[...] = jnp.zeros((1, 1), jnp.float32)
        n_ref[...] = jnp.zeros((1, 1), jnp.float32)

    s_ref[...] += s_part
    n_ref[...] += n_part


def kernel(loc_preds, loc_targets, cls_preds, cls_targets):
    del cls_preds
    B, A, K = loc_preds.shape
    total = B * A * K
    rows = total // 128
    anchors_per_row = 128 // K

    lp = loc_preds.reshape(rows, 128)
    lt = loc_targets.reshape(rows, 128)
    ct = cls_targets.reshape(rows, anchors_per_row)

    block_rows = 4096
    while rows % block_rows:
        block_rows //= 2
    grid = rows // block_rows

    s, n = pl.pallas_call(
        functools.partial(_ohem_reduce_kernel, anchors_per_row=anchors_per_row),
        grid=(grid,),
        in_specs=[
            pl.BlockSpec((block_rows, 128), lambda i: (i, 0)),
            pl.BlockSpec((block_rows, 128), lambda i: (i, 0)),
            pl.BlockSpec((block_rows, anchors_per_row), lambda i: (i, 0)),
        ],
        out_specs=[
            pl.BlockSpec((1, 1), lambda i: (0, 0)),
            pl.BlockSpec((1, 1), lambda i: (0, 0)),
        ],
        out_shape=[
            jax.ShapeDtypeStruct((1, 1), jnp.float32),
            jax.ShapeDtypeStruct((1, 1), jnp.float32),
        ],
    )(lp, lt, ct)

    loc_loss = s[0, 0]
    num_pos = n[0, 0]
    return 0.2 * loc_loss / num_pos


# speedup vs baseline: 2.0554x; 1.0276x over previous
import functools

import jax
import jax.numpy as jnp
from jax.experimental import pallas as pl


def _ohem_body(lp_ref, lt_ref, ct_ref, s_ref, n_ref, *, ta, chunks):
    i = pl.program_id(0)
    b = i // chunks
    a0 = (i % chunks) * ta

    diff = lp_ref[0] - lt_ref[0]
    diff_t = jnp.transpose(diff, (1, 0))

    absd = jnp.abs(diff_t)
    sl1 = jnp.where(absd < 1.0, 0.5 * diff_t * diff_t, absd - 0.5)

    pos = (ct_ref[pl.ds(b, 1), pl.ds(a0, ta)] > 0).astype(jnp.float32)

    s_part = jnp.sum(sl1 * pos).reshape(1, 1)
    n_part = jnp.sum(pos).reshape(1, 1)

    @pl.when(i == 0)
    def _init():
        s_ref[...] = jnp.zeros((1, 1), jnp.float32)
        n_ref[...] = jnp.zeros((1, 1), jnp.float32)

    s_ref[...] += s_part
    n_ref[...] += n_part


def kernel(loc_preds, loc_targets, cls_preds, cls_targets):
    del cls_preds
    B, A, K = loc_preds.shape

    ta = 8192
    while A % ta:
        ta //= 2
    chunks = A // ta
    grid = B * chunks

    body = functools.partial(_ohem_body, ta=ta, chunks=chunks)

    s, n = pl.pallas_call(
        body,
        grid=(grid,),
        in_specs=[
            pl.BlockSpec((1, ta, K), lambda i: (i // chunks, i % chunks, 0)),
            pl.BlockSpec((1, ta, K), lambda i: (i // chunks, i % chunks, 0)),
            pl.BlockSpec((B, A), lambda i: (0, 0)),
        ],
        out_specs=[
            pl.BlockSpec((1, 1), lambda i: (0, 0)),
            pl.BlockSpec((1, 1), lambda i: (0, 0)),
        ],
        out_shape=[
            jax.ShapeDtypeStruct((1, 1), jnp.float32),
            jax.ShapeDtypeStruct((1, 1), jnp.float32),
        ],
    )(loc_preds, loc_targets, cls_targets)

    loc_loss = s[0, 0]
    num_pos = n[0, 0]
    return 0.2 * loc_loss / num_pos


# speedup vs baseline: 2.0711x; 1.0077x over previous
import functools

import jax
import jax.numpy as jnp
from jax.experimental import pallas as pl


def _ohem_body(lp_ref, lt_ref, ct_ref, s_ref, n_ref, *, ta, chunks):
    i = pl.program_id(0)
    b = i // chunks
    a0 = (i % chunks) * ta

    diff = lp_ref[0] - lt_ref[0]
    diff_t = jnp.transpose(diff, (1, 0))

    absd = jnp.abs(diff_t)
    sl1 = jnp.where(absd < 1.0, 0.5 * diff_t * diff_t, absd - 0.5)

    pos = (ct_ref[pl.ds(b, 1), pl.ds(a0, ta)] > 0).astype(jnp.float32)

    s_part = jnp.sum(sl1 * pos).reshape(1, 1)
    n_part = jnp.sum(pos).reshape(1, 1)

    @pl.when(i == 0)
    def _init():
        s_ref[...] = jnp.zeros((1, 1), jnp.float32)
        n_ref[...] = jnp.zeros((1, 1), jnp.float32)

    s_ref[...] += s_part
    n_ref[...] += n_part


def kernel(loc_preds, loc_targets, cls_preds, cls_targets):
    del cls_preds
    B, A, K = loc_preds.shape

    ta = 16384
    while A % ta:
        ta //= 2
    chunks = A // ta
    grid = B * chunks

    body = functools.partial(_ohem_body, ta=ta, chunks=chunks)

    s, n = pl.pallas_call(
        body,
        grid=(grid,),
        in_specs=[
            pl.BlockSpec((1, ta, K), lambda i: (i // chunks, i % chunks, 0)),
            pl.BlockSpec((1, ta, K), lambda i: (i // chunks, i % chunks, 0)),
            pl.BlockSpec((B, A), lambda i: (0, 0)),
        ],
        out_specs=[
            pl.BlockSpec((1, 1), lambda i: (0, 0)),
            pl.BlockSpec((1, 1), lambda i: (0, 0)),
        ],
        out_shape=[
            jax.ShapeDtypeStruct((1, 1), jnp.float32),
            jax.ShapeDtypeStruct((1, 1), jnp.float32),
        ],
    )(loc_preds, loc_targets, cls_targets)

    loc_loss = s[0, 0]
    num_pos = n[0, 0]
    return 0.2 * loc_loss / num_pos
